# Initial kernel scaffold; baseline (speedup 1.0000x reference)
import jax
import jax.numpy as jnp
from jax.experimental import pallas as pl


def kernel(x, edge_index_atac, edge_index_rna, W1_atac, b1_atac, W1_rna, b1_rna, Wl_atac, Wr_atac, att_atac, bg_atac, Wl_rna, Wr_rna, att_rna, bg_rna, Wz_atac, bz_atac, Wz_rna, bz_rna, g1, beta1, g2, beta2):
    raise NotImplementedError("write your pallas kernel here")


# speedup vs baseline: 15.7571x; 15.7571x over previous
import functools

import jax
import jax.numpy as jnp
from jax import lax
from jax.experimental import pallas as pl
from jax.experimental.pallas import tpu as pltpu
from jax.experimental.pallas import tpu_sc as plsc

N = 10000
E = 320000
D = 128
DZ = 64
H = 8
DH = 16

NTILES = 16
LANES = 16

CH = 128
CPT = 162
EPT = CPT * CH
EPAD = NTILES * EPT
RPT = 632
NPAD = NTILES * RPT

_DUMP_SIZES = (128, 128, 128, 128, RPT - 4 * 128)


def _mesh():
  return plsc.VectorSubcoreMesh(
      core_axis_name="c", subcore_axis_name="s", num_cores=1)


def _zero_vec_buf(ref, rows, width):
  zv = jnp.zeros((LANES,), jnp.float32)

  def body(r, c):
    for j in range(width // LANES):
      ref[r, pl.ds(LANES * j, LANES)] = zv
    return c

  lax.fori_loop(0, rows, body, 0)


def _fill_rows16(ref, rows, value):
  vals = jnp.full((LANES,), value, jnp.float32)

  def body(r, c):
    ref[r, pl.ds(0, LANES)] = vals
    return c

  lax.fori_loop(0, rows, body, 0)


def _zero_shared(buf_v, acc_sh, row0):
  off = 0
  for sz in _DUMP_SIZES:
    pltpu.sync_copy(buf_v.at[pl.ds(0, sz)], acc_sh.at[pl.ds(row0 + off, sz)])
    off += sz


def _dump_shared(acc_sh, buf_v, out_hbm, row0):
  off = 0
  for sz in _DUMP_SIZES:
    pltpu.sync_copy(acc_sh.at[pl.ds(row0 + off, sz)], buf_v.at[pl.ds(0, sz)])
    pltpu.sync_copy(buf_v.at[pl.ds(0, sz)], out_hbm.at[pl.ds(row0 + off, sz)])
    off += sz


def _sc_degree():
  @functools.partial(
      pl.kernel,
      out_type=jax.ShapeDtypeStruct((NPAD, LANES), jnp.float32),
      mesh=_mesh(),
      compiler_params=pltpu.CompilerParams(use_tc_tiling_on_sc=False),
      scratch_types=[
          pltpu.VMEM((CH,), jnp.int32),
          pltpu.VMEM((CH, LANES), jnp.float32),
          pltpu.VMEM((128, LANES), jnp.float32),
          pltpu.VMEM_SHARED((NPAD, LANES), jnp.float32),
      ],
  )
  def k(dst_hbm, out_hbm, dst_v, ones_v, zbuf_v, acc_sh):
    sid = lax.axis_index("s")
    _fill_rows16(ones_v, CH, 1.0)
    _fill_rows16(zbuf_v, 128, 0.0)
    row0 = sid * RPT
    _zero_shared(zbuf_v, acc_sh, row0)
    plsc.subcore_barrier()

    ebase = sid * EPT

    def body(i, c):
      b = ebase + i * CH
      pltpu.sync_copy(dst_hbm.at[0, pl.ds(b, CH)], dst_v)
      pltpu.sync_copy(ones_v, acc_sh.at[dst_v], add=True)
      return c

    lax.fori_loop(0, CPT, body, 0)
    plsc.subcore_barrier()
    _dump_shared(acc_sh, zbuf_v, out_hbm, row0)

  return k


def _sc_scatter(width):
  @functools.partial(
      pl.kernel,
      out_type=jax.ShapeDtypeStruct((NPAD, width), jnp.float32),
      mesh=_mesh(),
      compiler_params=pltpu.CompilerParams(use_tc_tiling_on_sc=False),
      scratch_types=[
          pltpu.VMEM((CH,), jnp.int32),
          pltpu.VMEM((CH,), jnp.int32),
          pltpu.VMEM((CH, width), jnp.float32),
          pltpu.VMEM((128, width), jnp.float32),
          pltpu.VMEM_SHARED((NPAD, width), jnp.float32),
          pltpu.SemaphoreType.DMA,
      ],
  )
  def k(src_hbm, dst_hbm, rows_hbm, out_hbm, src_v, dst_v, rows_v, buf_v,
        acc_sh, sem):
    sid = lax.axis_index("s")
    _zero_vec_buf(buf_v, 128, width)
    row0 = sid * RPT
    _zero_shared(buf_v, acc_sh, row0)
    plsc.subcore_barrier()

    ebase = sid * EPT

    def body(i, c):
      b = ebase + i * CH
      pltpu.sync_copy(src_hbm.at[0, pl.ds(b, CH)], src_v)
      pltpu.sync_copy(dst_hbm.at[0, pl.ds(b, CH)], dst_v)
      pltpu.async_copy(rows_hbm.at[src_v], rows_v, sem).wait()
      pltpu.sync_copy(rows_v, acc_sh.at[dst_v], add=True)
      return c

    lax.fori_loop(0, CPT, body, 0)
    plsc.subcore_barrier()
    _dump_shared(acc_sh, buf_v, out_hbm, row0)

  return k


def _sc_gat_half():
  HH = H // 2
  WID = HH * DH

  @functools.partial(
      pl.kernel,
      out_type=(
          jax.ShapeDtypeStruct((NPAD, WID), jnp.float32),
          jax.ShapeDtypeStruct((NPAD, LANES), jnp.float32),
      ),
      mesh=_mesh(),
      compiler_params=pltpu.CompilerParams(use_tc_tiling_on_sc=False),
      scratch_types=[
          pltpu.VMEM((CH,), jnp.int32),
          pltpu.VMEM((CH,), jnp.int32),
          pltpu.VMEM((CH, WID), jnp.float32),
          pltpu.VMEM((CH, WID), jnp.float32),
          pltpu.VMEM((CH, LANES), jnp.float32),
          pltpu.VMEM((HH, DH), jnp.float32),
          pltpu.VMEM((128, WID), jnp.float32),
          pltpu.VMEM((128, LANES), jnp.float32),
          pltpu.VMEM_SHARED((NPAD, WID), jnp.float32),
          pltpu.VMEM_SHARED((NPAD, LANES), jnp.float32),
          pltpu.SemaphoreType.DMA,
          pltpu.SemaphoreType.DMA,
      ],
  )
  def k(src_hbm, dst_hbm, xl_hbm, xr_hbm, att_hbm, num_out, den_out,
        src_v, dst_v, xlr_v, xrr_v, ex_v, att_v, buf_v, buf16_v,
        num_sh, den_sh, sem_l, sem_r):
    sid = lax.axis_index("s")
    _zero_vec_buf(buf_v, 128, WID)
    _fill_rows16(buf16_v, 128, 0.0)
    row0 = sid * RPT
    _zero_shared(buf_v, num_sh, row0)
    _zero_shared(buf16_v, den_sh, row0)
    pltpu.sync_copy(att_hbm, att_v)
    plsc.subcore_barrier()

    att_regs = [att_v[h, pl.ds(0, DH)] for h in range(HH)]
    iota16 = lax.broadcasted_iota(jnp.int32, (LANES,), 0)
    perms = [jnp.bitwise_xor(iota16, sh) for sh in (1, 2, 4, 8)]
    hsplat = [jnp.full((LANES,), h, jnp.int32) for h in range(HH)]
    zv = jnp.zeros((LANES,), jnp.float32)

    ebase = sid * EPT

    def chunk(i, c):
      b = ebase + i * CH
      pltpu.sync_copy(src_hbm.at[0, pl.ds(b, CH)], src_v)
      pltpu.sync_copy(dst_hbm.at[0, pl.ds(b, CH)], dst_v)
      cp_l = pltpu.async_copy(xl_hbm.at[src_v], xlr_v, sem_l)
      cp_r = pltpu.async_copy(xr_hbm.at[dst_v], xrr_v, sem_r)
      cp_l.wait()
      cp_r.wait()

      def ebody(e, c2):
        acc = zv
        for h in range(HH):
          a = xlr_v[e, pl.ds(h * DH, DH)]
          bb = xrr_v[e, pl.ds(h * DH, DH)]
          s = a + bb
          lr = 0.6 * s + 0.4 * jnp.abs(s)
          cs = lr * att_regs[h]
          for p in perms:
            cs = cs + cs.at[p].get(mode="promise_in_bounds")
          acc = jnp.where(iota16 == h, cs, acc)
        ex = jnp.exp(acc)
        ex_v[e, pl.ds(0, LANES)] = ex
        for h in range(HH):
          exb = ex.at[hsplat[h]].get(mode="promise_in_bounds")
          xlr_v[e, pl.ds(h * DH, DH)] = xlr_v[e, pl.ds(h * DH, DH)] * exb
        return c2

      lax.fori_loop(0, CH, ebody, 0)

      pltpu.sync_copy(xlr_v, num_sh.at[dst_v], add=True)
      pltpu.sync_copy(ex_v, den_sh.at[dst_v], add=True)
      return c

    lax.fori_loop(0, CPT, chunk, 0)
    plsc.subcore_barrier()
    _dump_shared(num_sh, buf_v, num_out, row0)
    _dump_shared(den_sh, buf16_v, den_out, row0)

  return k


def _row_mask():
  return (lax.broadcasted_iota(jnp.int32, (NPAD, 1), 0) < N).astype(jnp.float32)


def _tc1_body(x_ref, w1_ref, dga_ref, dgr_ref,
              hsa_ref, hsr_ref, dia_ref, dir_ref):
  x = x_ref[...]
  for w, dg, hs, di in ((0, dga_ref, hsa_ref, dia_ref),
                        (1, dgr_ref, hsr_ref, dir_ref)):
    d = dg[:, 0:1]
    dinv = jnp.where(d > 0, lax.rsqrt(d), 0.0)
    di[...] = dinv
    hs[...] = jnp.dot(x, w1_ref[w], preferred_element_type=jnp.float32) * dinv


def _bn_silu(hp, g_ref, beta_ref):
  mask = _row_mask()
  hm = hp * mask
  mu = jnp.sum(hm, axis=0, keepdims=True) / N
  var = jnp.sum(hm * hp, axis=0, keepdims=True) / N - mu * mu
  hb = (hp - mu) * lax.rsqrt(var + 1e-5) * g_ref[...] + beta_ref[...]
  return hb * jax.nn.sigmoid(hb) * mask


def _tc2a_body(sa_ref, sr_ref, dia_ref, dir_ref, b1_ref, g1_ref, beta1_ref,
               h_ref):
  hp = (sa_ref[...] * dia_ref[...] + b1_ref[0]
        + sr_ref[...] * dir_ref[...] + b1_ref[1])
  h_ref[...] = _bn_silu(hp, g1_ref, beta1_ref)


def _tc2b_body(h_ref, wl_ref, wr_ref, *outs):
  h = h_ref[...]
  i = 0
  for m in range(2):
    for half in range(2):
      c0 = half * 64
      outs[i][...] = jnp.dot(h, wl_ref[m, :, c0:c0 + 64],
                             preferred_element_type=jnp.float32)
      outs[i + 1][...] = jnp.dot(h, wr_ref[m, :, c0:c0 + 64],
                                 preferred_element_type=jnp.float32)
      i += 2


def _tc3a_body(n00_ref, d00_ref, n01_ref, d01_ref,
               n10_ref, d10_ref, n11_ref, d11_ref,
               bg_ref, g2_ref, beta2_ref, h2_ref):
  hh = H // 2
  rh = lax.broadcasted_iota(jnp.int32, (hh, hh * DH), 0)
  rc = lax.broadcasted_iota(jnp.int32, (hh, hh * DH), 1)
  rep = (rh == rc // DH).astype(jnp.float32)
  halves = (((n00_ref, d00_ref), (n01_ref, d01_ref)),
            ((n10_ref, d10_ref), (n11_ref, d11_ref)))
  hp = 0.0
  for m in range(2):
    parts = []
    for half in range(2):
      num, den = halves[m][half]
      dd = jnp.dot(den[:, 0:hh], rep, preferred_element_type=jnp.float32)
      parts.append(num[...] / (dd + 1e-16))
    hp = hp + jnp.concatenate(parts, axis=1) + bg_ref[m]
  h2_ref[...] = _bn_silu(hp, g2_ref, beta2_ref)


def _tc3b_body(h2_ref, wz_ref, dia_ref, dir_ref, hza_ref, hzr_ref):
  h2 = h2_ref[...]
  hza_ref[...] = (jnp.dot(h2, wz_ref[0], preferred_element_type=jnp.float32)
                  * dia_ref[...])
  hzr_ref[...] = (jnp.dot(h2, wz_ref[1], preferred_element_type=jnp.float32)
                  * dir_ref[...])


def _tc4_body(s3a_ref, s3r_ref, dia_ref, dir_ref, bz_ref, z_ref):
  z_ref[...] = (s3a_ref[...] * dia_ref[...] + bz_ref[0]
                + s3r_ref[...] * dir_ref[...] + bz_ref[1])


def _sd(shape):
  return jax.ShapeDtypeStruct(shape, jnp.float32)


def kernel(x, edge_index_atac, edge_index_rna,
           W1_atac, b1_atac, W1_rna, b1_rna,
           Wl_atac, Wr_atac, att_atac, bg_atac,
           Wl_rna, Wr_rna, att_rna, bg_rna,
           Wz_atac, bz_atac, Wz_rna, bz_rna,
           g1, beta1, g2, beta2):
  xp = jnp.zeros((NPAD, D), jnp.float32).at[:N].set(x)
  loops = jnp.arange(N, dtype=jnp.int32)
  pad = jnp.full((EPAD - E - N,), N, jnp.int32)

  def mk(ei):
    return (jnp.concatenate([ei[0].astype(jnp.int32), loops, pad])
            .reshape(1, EPAD),
            jnp.concatenate([ei[1].astype(jnp.int32), loops, pad])
            .reshape(1, EPAD))

  sa, da = mk(edge_index_atac)
  sr, dr = mk(edge_index_rna)

  w1 = jnp.stack([W1_atac, W1_rna])
  b1 = jnp.stack([b1_atac, b1_rna])
  wl = jnp.stack([Wl_atac, Wl_rna])
  wr = jnp.stack([Wr_atac, Wr_rna])
  bg = jnp.stack([bg_atac, bg_rna])
  wz = jnp.stack([Wz_atac, Wz_rna])
  bz = jnp.stack([bz_atac, bz_rna])

  deg_k = _sc_degree()
  dga = deg_k(da)
  dr_0, _ = lax.optimization_barrier((dr, dga))
  dgr = deg_k(dr_0)

  hsa, hsr, dia, dirv = pl.pallas_call(
      _tc1_body,
      out_shape=(_sd((NPAD, D)), _sd((NPAD, D)),
                 _sd((NPAD, 1)), _sd((NPAD, 1))),
  )(xp, w1, dga, dgr)
  sc128 = _sc_scatter(D)
  s1a = sc128(sa, da, hsa)
  sr_, dr_, hsr_, _ = lax.optimization_barrier((sr, dr, hsr, s1a))
  s1r = sc128(sr_, dr_, hsr_)

  h = pl.pallas_call(
      _tc2a_body, out_shape=_sd((NPAD, D)),
  )(s1a, s1r, dia, dirv, b1, g1, beta1)
  xl00, xr00, xl01, xr01, xl10, xr10, xl11, xr11 = pl.pallas_call(
      _tc2b_body, out_shape=tuple(_sd((NPAD, DZ)) for _ in range(8)),
  )(h, wl, wr)
  gat_k = _sc_gat_half()
  atts = (att_atac[:4], att_atac[4:], att_rna[:4], att_rna[4:])
  gat_in = [(sa, da, xl00, xr00, atts[0]), (sa, da, xl01, xr01, atts[1]),
            (sr, dr, xl10, xr10, atts[2]), (sr, dr, xl11, xr11, atts[3])]
  nums, dens = [], []
  prev = None
  for (gs, gd, gxl, gxr, gatt) in gat_in:
    if prev is not None:
      gs, gd, gxl, gxr, _ = lax.optimization_barrier((gs, gd, gxl, gxr, prev))
    n_, d_ = gat_k(gs, gd, gxl, gxr, gatt)
    nums.append(n_)
    dens.append(d_)
    prev = n_

  h2 = pl.pallas_call(
      _tc3a_body, out_shape=_sd((NPAD, D)),
  )(nums[0], dens[0], nums[1], dens[1],
    nums[2], dens[2], nums[3], dens[3], bg, g2, beta2)
  hza, hzr = pl.pallas_call(
      _tc3b_body, out_shape=(_sd((NPAD, DZ)), _sd((NPAD, DZ))),
  )(h2, wz, dia, dirv)

  sc64 = _sc_scatter(DZ)
  s3a = sc64(sa, da, hza)
  sr3, dr3, hzr_, _ = lax.optimization_barrier((sr, dr, hzr, s3a))
  s3r = sc64(sr3, dr3, hzr_)
  z = pl.pallas_call(
      _tc4_body, out_shape=_sd((NPAD, DZ)),
  )(s3a, s3r, dia, dirv, bz)
  return z[:N]


# speedup vs baseline: 23.0727x; 1.4643x over previous
import functools

import jax
import jax.numpy as jnp
from jax import lax
from jax.experimental import pallas as pl
from jax.experimental.pallas import tpu as pltpu
from jax.experimental.pallas import tpu_sc as plsc

N = 10000
E = 320000
D = 128
DZ = 64
H = 8
DH = 16

NTILES = 16
LANES = 16

CH = 96
CPT = 216
EPT = CPT * CH
EPAD = NTILES * EPT
RPT = 632
NPAD = NTILES * RPT

_DUMP_SIZES = (128, 128, 128, 128, RPT - 4 * 128)


def _mesh():
  return plsc.VectorSubcoreMesh(
      core_axis_name="c", subcore_axis_name="s", num_cores=1)


def _zero_vec_buf(ref, rows, width):
  zv = jnp.zeros((LANES,), jnp.float32)

  def body(r, c):
    for j in range(width // LANES):
      ref[r, pl.ds(LANES * j, LANES)] = zv
    return c

  lax.fori_loop(0, rows, body, 0)


def _fill_rows16(ref, rows, value):
  vals = jnp.full((LANES,), value, jnp.float32)

  def body(r, c):
    ref[r, pl.ds(0, LANES)] = vals
    return c

  lax.fori_loop(0, rows, body, 0)


def _zero_shared(buf_v, acc_sh, row0):
  off = 0
  for sz in _DUMP_SIZES:
    pltpu.sync_copy(buf_v.at[pl.ds(0, sz)], acc_sh.at[pl.ds(row0 + off, sz)])
    off += sz


def _dump_shared(acc_sh, buf_v, out_hbm, row0):
  off = 0
  for sz in _DUMP_SIZES:
    pltpu.sync_copy(acc_sh.at[pl.ds(row0 + off, sz)], buf_v.at[pl.ds(0, sz)])
    pltpu.sync_copy(buf_v.at[pl.ds(0, sz)], out_hbm.at[pl.ds(row0 + off, sz)])
    off += sz


def _preload_idx(hbm3, vbuf, sid):
  def body(j, c):
    pltpu.sync_copy(hbm3.at[0, pl.ds(sid * CPT + j * 27, 27), :],
                    vbuf.at[pl.ds(j * 27, 27)])
    return c

  lax.fori_loop(0, CPT // 27, body, 0)


def _copy_idx_row(src2d, row, dst1d):
  for j in range(CH // LANES):
    dst1d[pl.ds(j * LANES, LANES)] = src2d[row, pl.ds(j * LANES, LANES)]


def _sc_degree():
  @functools.partial(
      pl.kernel,
      out_type=jax.ShapeDtypeStruct((NPAD, LANES), jnp.float32),
      mesh=_mesh(),
      compiler_params=pltpu.CompilerParams(use_tc_tiling_on_sc=False),
      scratch_types=[
          pltpu.VMEM((CPT, CH), jnp.int32),
          pltpu.VMEM((CH, LANES), jnp.float32),
          pltpu.VMEM((128, LANES), jnp.float32),
          pltpu.VMEM_SHARED((NPAD, LANES), jnp.float32),
      ],
  )
  def k(dst_hbm, out_hbm, dst_all, ones_v, zbuf_v, acc_sh):
    sid = lax.axis_index("s")
    _fill_rows16(ones_v, CH, 1.0)
    _fill_rows16(zbuf_v, 128, 0.0)
    row0 = sid * RPT
    _zero_shared(zbuf_v, acc_sh, row0)
    _preload_idx(dst_hbm, dst_all, sid)
    plsc.subcore_barrier()

    def body(i, c):
      pltpu.sync_copy(ones_v, acc_sh.at[dst_all.at[i]], add=True)
      return c

    lax.fori_loop(0, CPT, body, 0)
    plsc.subcore_barrier()
    _dump_shared(acc_sh, zbuf_v, out_hbm, row0)

  return k


def _sc_scatter(width):
  @functools.partial(
      pl.kernel,
      out_type=jax.ShapeDtypeStruct((NPAD, width), jnp.float32),
      mesh=_mesh(),
      compiler_params=pltpu.CompilerParams(use_tc_tiling_on_sc=False),
      scratch_types=[
          pltpu.VMEM((CPT, CH), jnp.int32),
          pltpu.VMEM((CPT, CH), jnp.int32),
          pltpu.VMEM((CH,), jnp.int32),
          pltpu.VMEM((CH,), jnp.int32),
          pltpu.VMEM((CH,), jnp.int32),
          pltpu.VMEM((CH,), jnp.int32),
          pltpu.VMEM((CH, width), jnp.float32),
          pltpu.VMEM((CH, width), jnp.float32),
          pltpu.VMEM((128, width), jnp.float32),
          pltpu.VMEM_SHARED((NPAD, width), jnp.float32),
          pltpu.SemaphoreType.DMA,
          pltpu.SemaphoreType.DMA,
      ],
  )
  def k(src_hbm, dst_hbm, rows_hbm, out_hbm, src_all, dst_all,
        si0, si1, di0, di1, rows_v0, rows_v1, buf_v, acc_sh, sem0, sem1):
    sid = lax.axis_index("s")
    _zero_vec_buf(buf_v, 128, width)
    row0 = sid * RPT
    _zero_shared(buf_v, acc_sh, row0)
    _preload_idx(src_hbm, src_all, sid)
    _preload_idx(dst_hbm, dst_all, sid)
    plsc.subcore_barrier()

    sis = (si0, si1)
    dis = (di0, di1)
    bufs = (rows_v0, rows_v1)
    sems = (sem0, sem1)

    _copy_idx_row(src_all, 0, si0)
    _copy_idx_row(dst_all, 0, di0)
    pltpu.async_copy(rows_hbm.at[si0], rows_v0, sem0)

    def body(i2, c):
      for b in (0, 1):
        ci = i2 * 2 + b

        @pl.when(ci + 1 < CPT)
        def _():
          _copy_idx_row(src_all, ci + 1, sis[1 - b])
          _copy_idx_row(dst_all, ci + 1, dis[1 - b])
          pltpu.async_copy(rows_hbm.at[sis[1 - b]], bufs[1 - b], sems[1 - b])

        pltpu.make_async_copy(rows_hbm.at[sis[b]], bufs[b], sems[b]).wait()
        pltpu.sync_copy(bufs[b], acc_sh.at[dis[b]], add=True)
      return c

    lax.fori_loop(0, CPT // 2, body, 0)
    plsc.subcore_barrier()
    _dump_shared(acc_sh, buf_v, out_hbm, row0)

  return k


def _sc_gat_half():
  HH = H // 2
  WID = HH * DH

  @functools.partial(
      pl.kernel,
      out_type=(
          jax.ShapeDtypeStruct((NPAD, WID), jnp.float32),
          jax.ShapeDtypeStruct((NPAD, LANES), jnp.float32),
      ),
      mesh=_mesh(),
      compiler_params=pltpu.CompilerParams(use_tc_tiling_on_sc=False),
      scratch_types=[
          pltpu.VMEM((CPT, CH), jnp.int32),
          pltpu.VMEM((CPT, CH), jnp.int32),
          pltpu.VMEM((CH,), jnp.int32),
          pltpu.VMEM((CH,), jnp.int32),
          pltpu.VMEM((CH,), jnp.int32),
          pltpu.VMEM((CH,), jnp.int32),
          pltpu.VMEM((CH, WID), jnp.float32),
          pltpu.VMEM((CH, WID), jnp.float32),
          pltpu.VMEM((CH, WID), jnp.float32),
          pltpu.VMEM((CH, WID), jnp.float32),
          pltpu.VMEM((CH, LANES), jnp.float32),
          pltpu.VMEM((HH, DH), jnp.float32),
          pltpu.VMEM((128, WID), jnp.float32),
          pltpu.VMEM((128, LANES), jnp.float32),
          pltpu.VMEM_SHARED((NPAD, WID), jnp.float32),
          pltpu.VMEM_SHARED((NPAD, LANES), jnp.float32),
          pltpu.SemaphoreType.DMA,
          pltpu.SemaphoreType.DMA,
          pltpu.SemaphoreType.DMA,
          pltpu.SemaphoreType.DMA,
      ],
  )
  def k(src_hbm, dst_hbm, xl_hbm, xr_hbm, att_hbm, num_out, den_out,
        src_all, dst_all, si0, si1, di0, di1,
        xl_v0, xl_v1, xr_v0, xr_v1, ex_v, att_v,
        buf_v, buf16_v, num_sh, den_sh, sl0, sl1, sr0, sr1):
    sid = lax.axis_index("s")
    _zero_vec_buf(buf_v, 128, WID)
    _fill_rows16(buf16_v, 128, 0.0)
    row0 = sid * RPT
    _zero_shared(buf_v, num_sh, row0)
    _zero_shared(buf16_v, den_sh, row0)
    pltpu.sync_copy(att_hbm, att_v)
    _preload_idx(src_hbm, src_all, sid)
    _preload_idx(dst_hbm, dst_all, sid)
    plsc.subcore_barrier()

    att_regs = [att_v[h, pl.ds(0, DH)] for h in range(HH)]
    iota16 = lax.broadcasted_iota(jnp.int32, (LANES,), 0)
    perms = [jnp.bitwise_xor(iota16, sh) for sh in (1, 2, 4, 8)]
    hsplat = [jnp.full((LANES,), h, jnp.int32) for h in range(HH)]
    zv = jnp.zeros((LANES,), jnp.float32)

    sis = (si0, si1)
    dis = (di0, di1)
    xls = (xl_v0, xl_v1)
    xrs = (xr_v0, xr_v1)
    lsems = (sl0, sl1)
    rsems = (sr0, sr1)

    _copy_idx_row(src_all, 0, si0)
    _copy_idx_row(dst_all, 0, di0)
    pltpu.async_copy(xl_hbm.at[si0], xl_v0, sl0)
    pltpu.async_copy(xr_hbm.at[di0], xr_v0, sr0)

    def chunk(i2, c):
      for b in (0, 1):
        ci = i2 * 2 + b
        xlr_v = xls[b]
        xrr_v = xrs[b]

        @pl.when(ci + 1 < CPT)
        def _():
          _copy_idx_row(src_all, ci + 1, sis[1 - b])
          _copy_idx_row(dst_all, ci + 1, dis[1 - b])
          pltpu.async_copy(xl_hbm.at[sis[1 - b]], xls[1 - b], lsems[1 - b])
          pltpu.async_copy(xr_hbm.at[dis[1 - b]], xrs[1 - b], rsems[1 - b])

        pltpu.make_async_copy(xl_hbm.at[sis[b]], xlr_v, lsems[b]).wait()
        pltpu.make_async_copy(xr_hbm.at[dis[b]], xrr_v, rsems[b]).wait()

        def ebody(e, c2):
          acc = zv
          for h in range(HH):
            a = xlr_v[e, pl.ds(h * DH, DH)]
            bb = xrr_v[e, pl.ds(h * DH, DH)]
            ss = a + bb
            lr = 0.6 * ss + 0.4 * jnp.abs(ss)
            cs = lr * att_regs[h]
            for pp in perms:
              cs = cs + cs.at[pp].get(mode="promise_in_bounds")
            acc = jnp.where(iota16 == h, cs, acc)
          ex = jnp.exp(acc)
          ex_v[e, pl.ds(0, LANES)] = ex
          for h in range(HH):
            exb = ex.at[hsplat[h]].get(mode="promise_in_bounds")
            xlr_v[e, pl.ds(h * DH, DH)] = xlr_v[e, pl.ds(h * DH, DH)] * exb
          return c2

        lax.fori_loop(0, CH, ebody, 0)

        pltpu.sync_copy(xlr_v, num_sh.at[dis[b]], add=True)
        pltpu.sync_copy(ex_v, den_sh.at[dis[b]], add=True)
      return c

    lax.fori_loop(0, CPT // 2, chunk, 0)
    plsc.subcore_barrier()
    _dump_shared(num_sh, buf_v, num_out, row0)
    _dump_shared(den_sh, buf16_v, den_out, row0)

  return k


def _row_mask():
  return (lax.broadcasted_iota(jnp.int32, (NPAD, 1), 0) < N).astype(jnp.float32)


def _tc1_body(x_ref, w1_ref, dga_ref, dgr_ref,
              hsa0_ref, hsa1_ref, hsr0_ref, hsr1_ref, dia_ref, dir_ref):
  x = x_ref[...]
  for w, dg, hs0, hs1, di in ((0, dga_ref, hsa0_ref, hsa1_ref, dia_ref),
                              (1, dgr_ref, hsr0_ref, hsr1_ref, dir_ref)):
    d = dg[:, 0:1]
    dinv = jnp.where(d > 0, lax.rsqrt(d), 0.0)
    di[...] = dinv
    hs0[...] = jnp.dot(x, w1_ref[w, :, 0:DZ],
                       preferred_element_type=jnp.float32) * dinv
    hs1[...] = jnp.dot(x, w1_ref[w, :, DZ:D],
                       preferred_element_type=jnp.float32) * dinv


def _bn_silu(hp, g_ref, beta_ref):
  mask = _row_mask()
  hm = hp * mask
  mu = jnp.sum(hm, axis=0, keepdims=True) / N
  var = jnp.sum(hm * hp, axis=0, keepdims=True) / N - mu * mu
  hb = (hp - mu) * lax.rsqrt(var + 1e-5) * g_ref[...] + beta_ref[...]
  return hb * jax.nn.sigmoid(hb) * mask


def _tc2a_body(sa0_ref, sa1_ref, sr0_ref, sr1_ref, dia_ref, dir_ref,
               b1_ref, g1_ref, beta1_ref, h_ref):
  da = dia_ref[...]
  dr = dir_ref[...]
  hp = jnp.concatenate(
      [sa0_ref[...] * da + sr0_ref[...] * dr,
       sa1_ref[...] * da + sr1_ref[...] * dr], axis=1)
  hp = hp + b1_ref[0] + b1_ref[1]
  h_ref[...] = _bn_silu(hp, g1_ref, beta1_ref)


def _tc2b_body(h_ref, wl_ref, wr_ref, *outs):
  h = h_ref[...]
  i = 0
  for m in range(2):
    for half in range(2):
      c0 = half * 64
      outs[i][...] = jnp.dot(h, wl_ref[m, :, c0:c0 + 64],
                             preferred_element_type=jnp.float32)
      outs[i + 1][...] = jnp.dot(h, wr_ref[m, :, c0:c0 + 64],
                                 preferred_element_type=jnp.float32)
      i += 2


def _tc3a_body(n00_ref, d00_ref, n01_ref, d01_ref,
               n10_ref, d10_ref, n11_ref, d11_ref,
               bg_ref, g2_ref, beta2_ref, h2_ref):
  hh = H // 2
  rh = lax.broadcasted_iota(jnp.int32, (hh, hh * DH), 0)
  rc = lax.broadcasted_iota(jnp.int32, (hh, hh * DH), 1)
  rep = (rh == rc // DH).astype(jnp.float32)
  halves = (((n00_ref, d00_ref), (n01_ref, d01_ref)),
            ((n10_ref, d10_ref), (n11_ref, d11_ref)))
  hp = 0.0
  for m in range(2):
    parts = []
    for half in range(2):
      num, den = halves[m][half]
      dd = jnp.dot(den[:, 0:hh], rep, preferred_element_type=jnp.float32)
      parts.append(num[...] / (dd + 1e-16))
    hp = hp + jnp.concatenate(parts, axis=1) + bg_ref[m]
  h2_ref[...] = _bn_silu(hp, g2_ref, beta2_ref)


def _tc3b_body(h2_ref, wz_ref, dia_ref, dir_ref, hza_ref, hzr_ref):
  h2 = h2_ref[...]
  hza_ref[...] = (jnp.dot(h2, wz_ref[0], preferred_element_type=jnp.float32)
                  * dia_ref[...])
  hzr_ref[...] = (jnp.dot(h2, wz_ref[1], preferred_element_type=jnp.float32)
                  * dir_ref[...])


def _tc4_body(s3a_ref, s3r_ref, dia_ref, dir_ref, bz_ref, z_ref):
  z_ref[...] = (s3a_ref[...] * dia_ref[...] + bz_ref[0]
                + s3r_ref[...] * dir_ref[...] + bz_ref[1])


def _sd(shape):
  return jax.ShapeDtypeStruct(shape, jnp.float32)


def kernel(x, edge_index_atac, edge_index_rna,
           W1_atac, b1_atac, W1_rna, b1_rna,
           Wl_atac, Wr_atac, att_atac, bg_atac,
           Wl_rna, Wr_rna, att_rna, bg_rna,
           Wz_atac, bz_atac, Wz_rna, bz_rna,
           g1, beta1, g2, beta2):
  xp = jnp.zeros((NPAD, D), jnp.float32).at[:N].set(x)
  loops = jnp.arange(N, dtype=jnp.int32)
  pad = jnp.full((EPAD - E - N,), N, jnp.int32)

  def mk(ei):
    return (jnp.concatenate([ei[0].astype(jnp.int32), loops, pad])
            .reshape(1, EPAD // CH, CH),
            jnp.concatenate([ei[1].astype(jnp.int32), loops, pad])
            .reshape(1, EPAD // CH, CH))

  sa, da = mk(edge_index_atac)
  sr, dr = mk(edge_index_rna)

  w1 = jnp.stack([W1_atac, W1_rna])
  b1 = jnp.stack([b1_atac, b1_rna])
  wl = jnp.stack([Wl_atac, Wl_rna])
  wr = jnp.stack([Wr_atac, Wr_rna])
  bg = jnp.stack([bg_atac, bg_rna])
  wz = jnp.stack([Wz_atac, Wz_rna])
  bz = jnp.stack([bz_atac, bz_rna])

  deg_k = _sc_degree()
  dga = deg_k(da)
  dr_0, _ = lax.optimization_barrier((dr, dga))
  dgr = deg_k(dr_0)

  hsa0, hsa1, hsr0, hsr1, dia, dirv = pl.pallas_call(
      _tc1_body,
      out_shape=(_sd((NPAD, DZ)), _sd((NPAD, DZ)),
                 _sd((NPAD, DZ)), _sd((NPAD, DZ)),
                 _sd((NPAD, 1)), _sd((NPAD, 1))),
  )(xp, w1, dga, dgr)
  sc64 = _sc_scatter(DZ)
  gcn1_in = [(sa, da, hsa0), (sa, da, hsa1), (sr, dr, hsr0), (sr, dr, hsr1)]
  s1 = []
  prev = None
  for (gs, gd, ghs) in gcn1_in:
    if prev is not None:
      gs, gd, ghs, _ = lax.optimization_barrier((gs, gd, ghs, prev))
    r = sc64(gs, gd, ghs)
    s1.append(r)
    prev = r
  s1a0, s1a1, s1r0, s1r1 = s1

  h = pl.pallas_call(
      _tc2a_body, out_shape=_sd((NPAD, D)),
  )(s1a0, s1a1, s1r0, s1r1, dia, dirv, b1, g1, beta1)
  xl00, xr00, xl01, xr01, xl10, xr10, xl11, xr11 = pl.pallas_call(
      _tc2b_body, out_shape=tuple(_sd((NPAD, DZ)) for _ in range(8)),
  )(h, wl, wr)
  gat_k = _sc_gat_half()
  atts = (att_atac[:4], att_atac[4:], att_rna[:4], att_rna[4:])
  gat_in = [(sa, da, xl00, xr00, atts[0]), (sa, da, xl01, xr01, atts[1]),
            (sr, dr, xl10, xr10, atts[2]), (sr, dr, xl11, xr11, atts[3])]
  nums, dens = [], []
  prev = None
  for (gs, gd, gxl, gxr, gatt) in gat_in:
    if prev is not None:
      gs, gd, gxl, gxr, _ = lax.optimization_barrier((gs, gd, gxl, gxr, prev))
    n_, d_ = gat_k(gs, gd, gxl, gxr, gatt)
    nums.append(n_)
    dens.append(d_)
    prev = n_

  h2 = pl.pallas_call(
      _tc3a_body, out_shape=_sd((NPAD, D)),
  )(nums[0], dens[0], nums[1], dens[1],
    nums[2], dens[2], nums[3], dens[3], bg, g2, beta2)
  hza, hzr = pl.pallas_call(
      _tc3b_body, out_shape=(_sd((NPAD, DZ)), _sd((NPAD, DZ))),
  )(h2, wz, dia, dirv)

  s3a = sc64(sa, da, hza)
  sr3, dr3, hzr_, _ = lax.optimization_barrier((sr, dr, hzr, s3a))
  s3r = sc64(sr3, dr3, hzr_)
  z = pl.pallas_call(
      _tc4_body, out_shape=_sd((NPAD, DZ)),
  )(s3a, s3r, dia, dirv, bz)
  return z[:N]


# speedup vs baseline: 23.6917x; 1.0268x over previous
import functools

import jax
import jax.numpy as jnp
from jax import lax
from jax.experimental import pallas as pl
from jax.experimental.pallas import tpu as pltpu
from jax.experimental.pallas import tpu_sc as plsc

N = 10000
E = 320000
D = 128
DZ = 64
H = 8
DH = 16

NTILES = 16
LANES = 16

CH = 96
CPT = 216
EPT = CPT * CH
EPAD = NTILES * EPT
RPT = 632
NPAD = NTILES * RPT

_DUMP_SIZES = (128, 128, 128, 128, RPT - 4 * 128)


def _mesh():
  return plsc.VectorSubcoreMesh(
      core_axis_name="c", subcore_axis_name="s", num_cores=1)


def _zero_vec_buf(ref, rows, width):
  zv = jnp.zeros((LANES,), jnp.float32)

  def body(r, c):
    for j in range(width // LANES):
      ref[r, pl.ds(LANES * j, LANES)] = zv
    return c

  lax.fori_loop(0, rows, body, 0)


def _fill_rows16(ref, rows, value):
  vals = jnp.full((LANES,), value, jnp.float32)

  def body(r, c):
    ref[r, pl.ds(0, LANES)] = vals
    return c

  lax.fori_loop(0, rows, body, 0)


def _zero_shared(buf_v, acc_sh, row0):
  off = 0
  for sz in _DUMP_SIZES:
    pltpu.sync_copy(buf_v.at[pl.ds(0, sz)], acc_sh.at[pl.ds(row0 + off, sz)])
    off += sz


def _dump_shared(acc_sh, buf_v, out_hbm, row0):
  off = 0
  for sz in _DUMP_SIZES:
    pltpu.sync_copy(acc_sh.at[pl.ds(row0 + off, sz)], buf_v.at[pl.ds(0, sz)])
    pltpu.sync_copy(buf_v.at[pl.ds(0, sz)], out_hbm.at[pl.ds(row0 + off, sz)])
    off += sz


def _preload_idx(hbm3, vbuf, sid):
  def body(j, c):
    pltpu.sync_copy(hbm3.at[0, pl.ds(sid * CPT + j * 27, 27), :],
                    vbuf.at[pl.ds(j * 27, 27)])
    return c

  lax.fori_loop(0, CPT // 27, body, 0)


def _copy_idx_row(src2d, row, dst1d):
  for j in range(CH // LANES):
    dst1d[pl.ds(j * LANES, LANES)] = src2d[row, pl.ds(j * LANES, LANES)]


def _sc_degree():
  @functools.partial(
      pl.kernel,
      out_type=jax.ShapeDtypeStruct((NPAD, LANES), jnp.float32),
      mesh=_mesh(),
      compiler_params=pltpu.CompilerParams(use_tc_tiling_on_sc=False),
      scratch_types=[
          pltpu.VMEM((CPT, CH), jnp.int32),
          pltpu.VMEM((CH, LANES), jnp.float32),
          pltpu.VMEM((128, LANES), jnp.float32),
          pltpu.VMEM_SHARED((NPAD, LANES), jnp.float32),
      ],
  )
  def k(dst_hbm, out_hbm, dst_all, ones_v, zbuf_v, acc_sh):
    sid = lax.axis_index("s")
    _fill_rows16(ones_v, CH, 1.0)
    _fill_rows16(zbuf_v, 128, 0.0)
    row0 = sid * RPT
    _zero_shared(zbuf_v, acc_sh, row0)
    _preload_idx(dst_hbm, dst_all, sid)
    plsc.subcore_barrier()

    def body(i, c):
      pltpu.sync_copy(ones_v, acc_sh.at[dst_all.at[i]], add=True)
      return c

    lax.fori_loop(0, CPT, body, 0)
    plsc.subcore_barrier()
    _dump_shared(acc_sh, zbuf_v, out_hbm, row0)

  return k


def _sc_scatter(width):
  @functools.partial(
      pl.kernel,
      out_type=jax.ShapeDtypeStruct((NPAD, width), jnp.float32),
      mesh=_mesh(),
      compiler_params=pltpu.CompilerParams(use_tc_tiling_on_sc=False),
      scratch_types=[
          pltpu.VMEM((CPT, CH), jnp.int32),
          pltpu.VMEM((CPT, CH), jnp.int32),
          pltpu.VMEM((CH,), jnp.int32),
          pltpu.VMEM((CH,), jnp.int32),
          pltpu.VMEM((CH,), jnp.int32),
          pltpu.VMEM((CH,), jnp.int32),
          pltpu.VMEM((CH, width), jnp.float32),
          pltpu.VMEM((CH, width), jnp.float32),
          pltpu.VMEM((128, width), jnp.float32),
          pltpu.VMEM_SHARED((NPAD, width), jnp.float32),
          pltpu.SemaphoreType.DMA,
          pltpu.SemaphoreType.DMA,
      ],
  )
  def k(src_hbm, dst_hbm, rows_hbm, out_hbm, src_all, dst_all,
        si0, si1, di0, di1, rows_v0, rows_v1, buf_v, acc_sh, sem0, sem1):
    sid = lax.axis_index("s")
    _zero_vec_buf(buf_v, 128, width)
    row0 = sid * RPT
    _zero_shared(buf_v, acc_sh, row0)
    _preload_idx(src_hbm, src_all, sid)
    _preload_idx(dst_hbm, dst_all, sid)
    plsc.subcore_barrier()

    sis = (si0, si1)
    dis = (di0, di1)
    bufs = (rows_v0, rows_v1)
    sems = (sem0, sem1)

    _copy_idx_row(src_all, 0, si0)
    _copy_idx_row(dst_all, 0, di0)
    pltpu.async_copy(rows_hbm.at[si0], rows_v0, sem0)

    def body(i2, c):
      for b in (0, 1):
        ci = i2 * 2 + b

        @pl.when(ci + 1 < CPT)
        def _():
          _copy_idx_row(src_all, ci + 1, sis[1 - b])
          _copy_idx_row(dst_all, ci + 1, dis[1 - b])
          pltpu.async_copy(rows_hbm.at[sis[1 - b]], bufs[1 - b], sems[1 - b])

        pltpu.make_async_copy(rows_hbm.at[sis[b]], bufs[b], sems[b]).wait()
        pltpu.sync_copy(bufs[b], acc_sh.at[dis[b]], add=True)
      return c

    lax.fori_loop(0, CPT // 2, body, 0)
    plsc.subcore_barrier()
    _dump_shared(acc_sh, buf_v, out_hbm, row0)

  return k


def _sc_gat_quarter():
  HP = 2
  WID = HP * DH

  @functools.partial(
      pl.kernel,
      out_type=(
          jax.ShapeDtypeStruct((2, NPAD, WID), jnp.float32),
          jax.ShapeDtypeStruct((2, NPAD, LANES), jnp.float32),
      ),
      mesh=plsc.VectorSubcoreMesh(core_axis_name="c", subcore_axis_name="s"),
      compiler_params=pltpu.CompilerParams(use_tc_tiling_on_sc=False),
      scratch_types=[
          pltpu.VMEM((CPT, CH), jnp.int32),
          pltpu.VMEM((CPT, CH), jnp.int32),
          pltpu.VMEM((CH,), jnp.int32),
          pltpu.VMEM((CH,), jnp.int32),
          pltpu.VMEM((CH,), jnp.int32),
          pltpu.VMEM((CH,), jnp.int32),
          pltpu.VMEM((CH, WID), jnp.float32),
          pltpu.VMEM((CH, WID), jnp.float32),
          pltpu.VMEM((CH, WID), jnp.float32),
          pltpu.VMEM((CH, WID), jnp.float32),
          pltpu.VMEM((CH, LANES), jnp.float32),
          pltpu.VMEM((HP, DH), jnp.float32),
          pltpu.VMEM((128, WID), jnp.float32),
          pltpu.VMEM((128, LANES), jnp.float32),
          pltpu.VMEM_SHARED((NPAD, WID), jnp.float32),
          pltpu.VMEM_SHARED((NPAD, LANES), jnp.float32),
          pltpu.SemaphoreType.DMA,
          pltpu.SemaphoreType.DMA,
          pltpu.SemaphoreType.DMA,
          pltpu.SemaphoreType.DMA,
      ],
  )
  def k(src_hbm, dst_hbm, xl_hbm, xr_hbm, att_hbm, num_out, den_out,
        src_all, dst_all, si0, si1, di0, di1,
        xl_v0, xl_v1, xr_v0, xr_v1, ex_v, att_v,
        buf_v, buf16_v, num_sh, den_sh, sl0, sl1, sr0, sr1):
    cid = lax.axis_index("c")
    sid = lax.axis_index("s")
    _zero_vec_buf(buf_v, 128, WID)
    _fill_rows16(buf16_v, 128, 0.0)
    row0 = sid * RPT
    _zero_shared(buf_v, num_sh, row0)
    _zero_shared(buf16_v, den_sh, row0)
    pltpu.sync_copy(att_hbm.at[pl.ds(cid * HP, HP)], att_v)
    _preload_idx(src_hbm, src_all, sid)
    _preload_idx(dst_hbm, dst_all, sid)
    plsc.subcore_barrier()

    att_regs = [att_v[h, pl.ds(0, DH)] for h in range(HP)]
    iota16 = lax.broadcasted_iota(jnp.int32, (LANES,), 0)
    perms = [jnp.bitwise_xor(iota16, sh) for sh in (1, 2, 4, 8)]
    hsplat = [jnp.full((LANES,), h, jnp.int32) for h in range(HP)]
    zv = jnp.zeros((LANES,), jnp.float32)

    sis = (si0, si1)
    dis = (di0, di1)
    xls = (xl_v0, xl_v1)
    xrs = (xr_v0, xr_v1)
    lsems = (sl0, sl1)
    rsems = (sr0, sr1)
    xl_t = xl_hbm.at[cid]
    xr_t = xr_hbm.at[cid]

    _copy_idx_row(src_all, 0, si0)
    _copy_idx_row(dst_all, 0, di0)
    pltpu.async_copy(xl_t.at[si0], xl_v0, sl0)
    pltpu.async_copy(xr_t.at[di0], xr_v0, sr0)

    def chunk(i2, c):
      for b in (0, 1):
        ci = i2 * 2 + b
        xlr_v = xls[b]
        xrr_v = xrs[b]

        @pl.when(ci + 1 < CPT)
        def _():
          _copy_idx_row(src_all, ci + 1, sis[1 - b])
          _copy_idx_row(dst_all, ci + 1, dis[1 - b])
          pltpu.async_copy(xl_t.at[sis[1 - b]], xls[1 - b], lsems[1 - b])
          pltpu.async_copy(xr_t.at[dis[1 - b]], xrs[1 - b], rsems[1 - b])

        pltpu.make_async_copy(xl_t.at[sis[b]], xlr_v, lsems[b]).wait()
        pltpu.make_async_copy(xr_t.at[dis[b]], xrr_v, rsems[b]).wait()

        def ebody(e, c2):
          acc = zv
          for h in range(HP):
            a = xlr_v[e, pl.ds(h * DH, DH)]
            bb = xrr_v[e, pl.ds(h * DH, DH)]
            ss = a + bb
            lr = 0.6 * ss + 0.4 * jnp.abs(ss)
            cs = lr * att_regs[h]
            for pp in perms:
              cs = cs + cs.at[pp].get(mode="promise_in_bounds")
            acc = jnp.where(iota16 == h, cs, acc)
          ex = jnp.exp(acc)
          ex_v[e, pl.ds(0, LANES)] = ex
          for h in range(HP):
            exb = ex.at[hsplat[h]].get(mode="promise_in_bounds")
            xlr_v[e, pl.ds(h * DH, DH)] = xlr_v[e, pl.ds(h * DH, DH)] * exb
          return c2

        lax.fori_loop(0, CH, ebody, 0)

        pltpu.sync_copy(xlr_v, num_sh.at[dis[b]], add=True)
        pltpu.sync_copy(ex_v, den_sh.at[dis[b]], add=True)
      return c

    lax.fori_loop(0, CPT // 2, chunk, 0)
    plsc.subcore_barrier()
    _dump_shared(num_sh, buf_v, num_out.at[cid], row0)
    _dump_shared(den_sh, buf16_v, den_out.at[cid], row0)

  return k


def _row_mask():
  return (lax.broadcasted_iota(jnp.int32, (NPAD, 1), 0) < N).astype(jnp.float32)


def _tc1_body(x_ref, w1_ref, dga_ref, dgr_ref,
              hsa0_ref, hsa1_ref, hsr0_ref, hsr1_ref, dia_ref, dir_ref):
  x = x_ref[...]
  for w, dg, hs0, hs1, di in ((0, dga_ref, hsa0_ref, hsa1_ref, dia_ref),
                              (1, dgr_ref, hsr0_ref, hsr1_ref, dir_ref)):
    d = dg[:, 0:1]
    dinv = jnp.where(d > 0, lax.rsqrt(d), 0.0)
    di[...] = dinv
    hs0[...] = jnp.dot(x, w1_ref[w, :, 0:DZ],
                       preferred_element_type=jnp.float32) * dinv
    hs1[...] = jnp.dot(x, w1_ref[w, :, DZ:D],
                       preferred_element_type=jnp.float32) * dinv


def _bn_silu(hp, g_ref, beta_ref):
  mask = _row_mask()
  hm = hp * mask
  mu = jnp.sum(hm, axis=0, keepdims=True) / N
  var = jnp.sum(hm * hp, axis=0, keepdims=True) / N - mu * mu
  hb = (hp - mu) * lax.rsqrt(var + 1e-5) * g_ref[...] + beta_ref[...]
  return hb * jax.nn.sigmoid(hb) * mask


def _tc2a_body(sa0_ref, sa1_ref, sr0_ref, sr1_ref, dia_ref, dir_ref,
               b1_ref, g1_ref, beta1_ref, h_ref):
  da = dia_ref[...]
  dr = dir_ref[...]
  hp = jnp.concatenate(
      [sa0_ref[...] * da + sr0_ref[...] * dr,
       sa1_ref[...] * da + sr1_ref[...] * dr], axis=1)
  hp = hp + b1_ref[0] + b1_ref[1]
  h_ref[...] = _bn_silu(hp, g1_ref, beta1_ref)


def _tc2b_body(h_ref, w_ref, out0_ref, out1_ref):
  h = h_ref[...]
  for half, out in ((0, out0_ref), (1, out1_ref)):
    for q in range(2):
      c0 = half * 64 + q * 32
      out[q] = jnp.dot(h, w_ref[:, c0:c0 + 32],
                       preferred_element_type=jnp.float32)


def _tc3a_mod_body(n0_ref, d0_ref, n1_ref, d1_ref, bg_ref, hp_ref):
  rh = lax.broadcasted_iota(jnp.int32, (2, 2 * DH), 0)
  rc = lax.broadcasted_iota(jnp.int32, (2, 2 * DH), 1)
  rep = (rh == rc // DH).astype(jnp.float32)
  parts = []
  for num, den in ((n0_ref, d0_ref), (n1_ref, d1_ref)):
    for q in range(2):
      dd = jnp.dot(den[q, :, 0:2], rep, preferred_element_type=jnp.float32)
      parts.append(num[q] / (dd + 1e-16))
  hp_ref[...] = jnp.concatenate(parts, axis=1) + bg_ref[...]


def _tc3a_bn_body(hpa_ref, hpr_ref, g2_ref, beta2_ref, h2_ref):
  h2_ref[...] = _bn_silu(hpa_ref[...] + hpr_ref[...], g2_ref, beta2_ref)


def _tc3b_body(h2_ref, wz_ref, dia_ref, dir_ref, hza_ref, hzr_ref):
  h2 = h2_ref[...]
  hza_ref[...] = (jnp.dot(h2, wz_ref[0], preferred_element_type=jnp.float32)
                  * dia_ref[...])
  hzr_ref[...] = (jnp.dot(h2, wz_ref[1], preferred_element_type=jnp.float32)
                  * dir_ref[...])


def _tc4_body(s3a_ref, s3r_ref, dia_ref, dir_ref, bz_ref, z_ref):
  z_ref[...] = (s3a_ref[...] * dia_ref[...] + bz_ref[0]
                + s3r_ref[...] * dir_ref[...] + bz_ref[1])


def _sd(shape):
  return jax.ShapeDtypeStruct(shape, jnp.float32)


def kernel(x, edge_index_atac, edge_index_rna,
           W1_atac, b1_atac, W1_rna, b1_rna,
           Wl_atac, Wr_atac, att_atac, bg_atac,
           Wl_rna, Wr_rna, att_rna, bg_rna,
           Wz_atac, bz_atac, Wz_rna, bz_rna,
           g1, beta1, g2, beta2):
  xp = jnp.zeros((NPAD, D), jnp.float32).at[:N].set(x)
  loops = jnp.arange(N, dtype=jnp.int32)
  pad = jnp.full((EPAD - E - N,), N, jnp.int32)

  def mk(ei):
    return (jnp.concatenate([ei[0].astype(jnp.int32), loops, pad])
            .reshape(1, EPAD // CH, CH),
            jnp.concatenate([ei[1].astype(jnp.int32), loops, pad])
            .reshape(1, EPAD // CH, CH))

  sa, da = mk(edge_index_atac)
  sr, dr = mk(edge_index_rna)

  w1 = jnp.stack([W1_atac, W1_rna])
  b1 = jnp.stack([b1_atac, b1_rna])
  wl = jnp.stack([Wl_atac, Wl_rna])
  wr = jnp.stack([Wr_atac, Wr_rna])
  bg = jnp.stack([bg_atac, bg_rna])
  wz = jnp.stack([Wz_atac, Wz_rna])
  bz = jnp.stack([bz_atac, bz_rna])

  deg_k = _sc_degree()
  dga = deg_k(da)
  dr_0, _ = lax.optimization_barrier((dr, dga))
  dgr = deg_k(dr_0)

  hsa0, hsa1, hsr0, hsr1, dia, dirv = pl.pallas_call(
      _tc1_body,
      out_shape=(_sd((NPAD, DZ)), _sd((NPAD, DZ)),
                 _sd((NPAD, DZ)), _sd((NPAD, DZ)),
                 _sd((NPAD, 1)), _sd((NPAD, 1))),
  )(xp, w1, dga, dgr)
  sc64 = _sc_scatter(DZ)
  gcn1_in = [(sa, da, hsa0), (sa, da, hsa1), (sr, dr, hsr0), (sr, dr, hsr1)]
  s1 = []
  prev = None
  for (gs, gd, ghs) in gcn1_in:
    if prev is not None:
      gs, gd, ghs, _ = lax.optimization_barrier((gs, gd, ghs, prev))
    r = sc64(gs, gd, ghs)
    s1.append(r)
    prev = r
  s1a0, s1a1, s1r0, s1r1 = s1

  h = pl.pallas_call(
      _tc2a_body, out_shape=_sd((NPAD, D)),
  )(s1a0, s1a1, s1r0, s1r1, dia, dirv, b1, g1, beta1)
  tc2b_out = (_sd((2, NPAD, 32)), _sd((2, NPAD, 32)))
  xl00, xl01 = pl.pallas_call(_tc2b_body, out_shape=tc2b_out)(h, Wl_atac)
  xr00, xr01 = pl.pallas_call(_tc2b_body, out_shape=tc2b_out)(h, Wr_atac)
  xl10, xl11 = pl.pallas_call(_tc2b_body, out_shape=tc2b_out)(h, Wl_rna)
  xr10, xr11 = pl.pallas_call(_tc2b_body, out_shape=tc2b_out)(h, Wr_rna)
  gat_k = _sc_gat_quarter()
  atts = (att_atac[:4], att_atac[4:], att_rna[:4], att_rna[4:])
  gat_in = [(sa, da, xl00, xr00, atts[0]), (sa, da, xl01, xr01, atts[1]),
            (sr, dr, xl10, xr10, atts[2]), (sr, dr, xl11, xr11, atts[3])]
  nums, dens = [], []
  prev = None
  for (gs, gd, gxl, gxr, gatt) in gat_in:
    if prev is not None:
      gs, gd, gxl, gxr, _ = lax.optimization_barrier((gs, gd, gxl, gxr, prev))
    n_, d_ = gat_k(gs, gd, gxl, gxr, gatt)
    nums.append(n_)
    dens.append(d_)
    prev = n_

  hpa = pl.pallas_call(
      _tc3a_mod_body, out_shape=_sd((NPAD, D)),
  )(nums[0], dens[0], nums[1], dens[1], bg_atac)
  hpr = pl.pallas_call(
      _tc3a_mod_body, out_shape=_sd((NPAD, D)),
  )(nums[2], dens[2], nums[3], dens[3], bg_rna)
  h2 = pl.pallas_call(
      _tc3a_bn_body, out_shape=_sd((NPAD, D)),
  )(hpa, hpr, g2, beta2)
  hza, hzr = pl.pallas_call(
      _tc3b_body, out_shape=(_sd((NPAD, DZ)), _sd((NPAD, DZ))),
  )(h2, wz, dia, dirv)

  s3a = sc64(sa, da, hza)
  sr3, dr3, hzr_, _ = lax.optimization_barrier((sr, dr, hzr, s3a))
  s3r = sc64(sr3, dr3, hzr_)
  z = pl.pallas_call(
      _tc4_body, out_shape=_sd((NPAD, DZ)),
  )(s3a, s3r, dia, dirv, bz)
  return z[:N]
